# baseline (speedup 1.0000x reference)
import dataclasses
import functools

import jax
import jax.numpy as jnp
from jax import lax
from jax.experimental import pallas as pl
from jax.experimental.pallas import tpu as pltpu
from jax.experimental.pallas import tpu_sc as plsc

N = 10000
E = 320000
D = 128

NUM_CORES = 2
NUM_SUBCORES = 16
NUM_WORKERS = NUM_CORES * NUM_SUBCORES
CHUNK = 320
NUM_CHUNKS = 32
EDGES_PER_WORKER = CHUNK * NUM_CHUNKS
E_PAD = EDGES_PER_WORKER * NUM_WORKERS
N_PAD = 10240
ROWS_PER_SUBCORE = N_PAD // NUM_SUBCORES
LANES = 16


def _linear_body(x_ref, wt_ref, b_ref, out_ref):
    out_ref[...] = (
        jnp.dot(x_ref[...], wt_ref[...], preferred_element_type=jnp.float32)
        + b_ref[...]
    )


def _tc_linear(x, wt, b2d):
    return pl.pallas_call(
        _linear_body,
        out_shape=jax.ShapeDtypeStruct((N, D), jnp.float32),
    )(x, wt, b2d)


def _add_tanh_body(p0_ref, p1_ref, out_ref):
    out_ref[...] = jnp.tanh(p0_ref[...] + p1_ref[...])


def _tc_add_tanh(p0, p1):
    blk = 2000
    return pl.pallas_call(
        _add_tanh_body,
        grid=(N // blk,),
        in_specs=[
            pl.BlockSpec((blk, D), lambda i: (i, 0)),
            pl.BlockSpec((blk, D), lambda i: (i, 0)),
        ],
        out_specs=pl.BlockSpec((blk, D), lambda i: (i, 0)),
        out_shape=jax.ShapeDtypeStruct((N, D), jnp.float32),
    )(p0, p1)


def _sc_aggregate(support, src, dst, adj, zeros):
    mesh = plsc.VectorSubcoreMesh(
        core_axis_name="c", subcore_axis_name="s", num_cores=NUM_CORES
    )
    cp = pltpu.CompilerParams()
    if "needs_layout_passes" in pltpu.CompilerParams.__dataclass_fields__:
        cp = dataclasses.replace(cp, needs_layout_passes=False)

    @functools.partial(
        pl.kernel,
        compiler_params=cp,
        out_type=jax.ShapeDtypeStruct((NUM_CORES, N_PAD, D), jnp.float32),
        mesh=mesh,
        scratch_types=[
            pltpu.VMEM_SHARED((N_PAD, D), jnp.float32),
            pltpu.VMEM((CHUNK,), jnp.int32),
            pltpu.VMEM((CHUNK,), jnp.int32),
            pltpu.VMEM((CHUNK,), jnp.float32),
            pltpu.VMEM((CHUNK, D), jnp.float32),
            pltpu.SemaphoreType.DMA,
        ],
    )
    def agg_kernel(
        support_hbm, src_hbm, dst_hbm, adj_hbm, zeros_hbm, out_hbm,
        acc, src_v, dst_v, adj_v, rows_v, sem,
    ):
        core = lax.axis_index("c")
        sub = lax.axis_index("s")
        worker = core * NUM_SUBCORES + sub
        row0 = sub * ROWS_PER_SUBCORE

        pltpu.sync_copy(zeros_hbm, acc.at[pl.ds(row0, ROWS_PER_SUBCORE)])
        plsc.subcore_barrier()

        @pl.loop(0, NUM_CHUNKS)
        def _(c):
            pltpu.sync_copy(src_hbm.at[worker, c], src_v)
            pltpu.sync_copy(dst_hbm.at[worker, c], dst_v)
            pltpu.sync_copy(adj_hbm.at[worker, c], adj_v)
            pltpu.async_copy(support_hbm.at[src_v], rows_v, sem).wait()

            @pl.loop(0, CHUNK)
            def _(r):
                ir = lax.broadcast_in_dim(r, (LANES,), ())
                a = plsc.load_gather(adj_v, [ir])
                for j in range(D // LANES):
                    sl = pl.ds(j * LANES, LANES)
                    rows_v[r, sl] = rows_v[r, sl] * a

            pltpu.sync_copy(rows_v, acc.at[dst_v], add=True)

        plsc.subcore_barrier()
        pltpu.sync_copy(
            acc.at[pl.ds(row0, ROWS_PER_SUBCORE)],
            out_hbm.at[core, pl.ds(row0, ROWS_PER_SUBCORE)],
        )

    return agg_kernel(support, src, dst, adj, zeros)


@jax.jit
def kernel(x, edge_index, adj_values, W, b):
    pad = E_PAD - E
    src = jnp.concatenate(
        [edge_index[1].astype(jnp.int32), jnp.zeros((pad,), jnp.int32)]
    ).reshape(NUM_WORKERS, NUM_CHUNKS, CHUNK)
    dst = jnp.concatenate(
        [edge_index[0].astype(jnp.int32), jnp.full((pad,), N, jnp.int32)]
    ).reshape(NUM_WORKERS, NUM_CHUNKS, CHUNK)
    adj = jnp.concatenate(
        [adj_values, jnp.zeros((pad,), jnp.float32)]
    ).reshape(NUM_WORKERS, NUM_CHUNKS, CHUNK)
    wt = W.T
    b2d = b.reshape(1, D)
    support = _tc_linear(x, wt, b2d)
    zeros = jnp.zeros((ROWS_PER_SUBCORE, D), jnp.float32)
    partials = _sc_aggregate(support, src, dst, adj, zeros)
    return _tc_add_tanh(partials[0, :N], partials[1, :N])


# speedup vs baseline: 1.0011x; 1.0011x over previous
import dataclasses
import functools

import jax
import jax.numpy as jnp
from jax import lax
from jax.experimental import pallas as pl
from jax.experimental.pallas import tpu as pltpu
from jax.experimental.pallas import tpu_sc as plsc

N = 10000
E = 320000
D = 128

NUM_CORES = 2
NUM_SUBCORES = 16
NUM_WORKERS = NUM_CORES * NUM_SUBCORES
CHUNK = 200
NUM_CHUNKS = 51
EDGES_PER_WORKER = CHUNK * NUM_CHUNKS
E_PAD = EDGES_PER_WORKER * NUM_WORKERS
N_PAD = 10240
ROWS_PER_SUBCORE = N_PAD // NUM_SUBCORES
LANES = 16


def _linear_body(x_ref, wt_ref, b_ref, out_ref):
    out_ref[...] = (
        jnp.dot(x_ref[...], wt_ref[...], preferred_element_type=jnp.float32)
        + b_ref[...]
    )


def _tc_linear(x, wt, b2d):
    return pl.pallas_call(
        _linear_body,
        out_shape=jax.ShapeDtypeStruct((N, D), jnp.float32),
    )(x, wt, b2d)


def _add_tanh_body(p0_ref, p1_ref, out_ref):
    out_ref[...] = jnp.tanh(p0_ref[...] + p1_ref[...])


def _tc_add_tanh(p0, p1):
    blk = 2000
    return pl.pallas_call(
        _add_tanh_body,
        grid=(N // blk,),
        in_specs=[
            pl.BlockSpec((blk, D), lambda i: (i, 0)),
            pl.BlockSpec((blk, D), lambda i: (i, 0)),
        ],
        out_specs=pl.BlockSpec((blk, D), lambda i: (i, 0)),
        out_shape=jax.ShapeDtypeStruct((N, D), jnp.float32),
    )(p0, p1)


def _sc_aggregate(support, src, dst, adj, zeros):
    mesh = plsc.VectorSubcoreMesh(
        core_axis_name="c", subcore_axis_name="s", num_cores=NUM_CORES
    )
    cp = pltpu.CompilerParams()
    if "needs_layout_passes" in pltpu.CompilerParams.__dataclass_fields__:
        cp = dataclasses.replace(cp, needs_layout_passes=False)

    @functools.partial(
        pl.kernel,
        compiler_params=cp,
        out_type=jax.ShapeDtypeStruct((NUM_CORES, N_PAD, D), jnp.float32),
        mesh=mesh,
        scratch_types=[
            pltpu.VMEM_SHARED((N_PAD, D), jnp.float32),
            pltpu.VMEM((CHUNK,), jnp.int32),
            pltpu.VMEM((CHUNK,), jnp.int32),
            pltpu.VMEM((CHUNK,), jnp.float32),
            pltpu.VMEM((CHUNK, D), jnp.float32),
            pltpu.SemaphoreType.DMA,
        ],
    )
    def agg_kernel(
        support_hbm, src_hbm, dst_hbm, adj_hbm, zeros_hbm, out_hbm,
        acc, src_v, dst_v, adj_v, rows_v, sem,
    ):
        core = lax.axis_index("c")
        sub = lax.axis_index("s")
        worker = core * NUM_SUBCORES + sub
        row0 = sub * ROWS_PER_SUBCORE

        pltpu.sync_copy(zeros_hbm, acc.at[pl.ds(row0, ROWS_PER_SUBCORE)])
        plsc.subcore_barrier()

        @pl.loop(0, NUM_CHUNKS)
        def _(c):
            pltpu.sync_copy(src_hbm.at[worker, c], src_v)
            pltpu.sync_copy(dst_hbm.at[worker, c], dst_v)
            pltpu.sync_copy(adj_hbm.at[worker, c], adj_v)
            pltpu.async_copy(support_hbm.at[src_v], rows_v, sem).wait()

            @pl.loop(0, CHUNK)
            def _(r):
                ir = lax.broadcast_in_dim(r, (LANES,), ())
                a = plsc.load_gather(adj_v, [ir])
                for j in range(D // LANES):
                    sl = pl.ds(j * LANES, LANES)
                    rows_v[r, sl] = rows_v[r, sl] * a

            pltpu.sync_copy(rows_v, acc.at[dst_v], add=True)

        plsc.subcore_barrier()
        pltpu.sync_copy(
            acc.at[pl.ds(row0, ROWS_PER_SUBCORE)],
            out_hbm.at[core, pl.ds(row0, ROWS_PER_SUBCORE)],
        )

    return agg_kernel(support, src, dst, adj, zeros)


@jax.jit
def kernel(x, edge_index, adj_values, W, b):
    pad = E_PAD - E
    src = jnp.concatenate(
        [edge_index[1].astype(jnp.int32), jnp.zeros((pad,), jnp.int32)]
    ).reshape(NUM_WORKERS, NUM_CHUNKS, CHUNK)
    dst = jnp.concatenate(
        [edge_index[0].astype(jnp.int32), jnp.full((pad,), N, jnp.int32)]
    ).reshape(NUM_WORKERS, NUM_CHUNKS, CHUNK)
    adj = jnp.concatenate(
        [adj_values, jnp.zeros((pad,), jnp.float32)]
    ).reshape(NUM_WORKERS, NUM_CHUNKS, CHUNK)
    wt = W.T
    b2d = b.reshape(1, D)
    support = _tc_linear(x, wt, b2d)
    zeros = jnp.zeros((ROWS_PER_SUBCORE, D), jnp.float32)
    partials = _sc_aggregate(support, src, dst, adj, zeros)
    return _tc_add_tanh(partials[0, :N], partials[1, :N])


# speedup vs baseline: 1.1895x; 1.1882x over previous
import dataclasses
import functools

import jax
import jax.numpy as jnp
from jax import lax
from jax.experimental import pallas as pl
from jax.experimental.pallas import tpu as pltpu
from jax.experimental.pallas import tpu_sc as plsc

N = 10000
E = 320000
D = 128

NUM_CORES = 2
NUM_SUBCORES = 16
NUM_WORKERS = NUM_CORES * NUM_SUBCORES
CHUNK = 160
NUM_CHUNKS = 64
EDGES_PER_WORKER = CHUNK * NUM_CHUNKS
E_PAD = EDGES_PER_WORKER * NUM_WORKERS
N_PAD = 10240
ROWS_PER_SUBCORE = N_PAD // NUM_SUBCORES
LANES = 16


def _linear_body(x_ref, wt_ref, b_ref, out_ref):
    out_ref[...] = (
        jnp.dot(x_ref[...], wt_ref[...], preferred_element_type=jnp.float32)
        + b_ref[...]
    )


def _tc_linear(x, wt, b2d):
    return pl.pallas_call(
        _linear_body,
        out_shape=jax.ShapeDtypeStruct((N, D), jnp.float32),
    )(x, wt, b2d)


def _add_tanh_body(p0_ref, p1_ref, out_ref):
    out_ref[...] = jnp.tanh(p0_ref[...] + p1_ref[...])


def _tc_add_tanh(p0, p1):
    blk = 2000
    return pl.pallas_call(
        _add_tanh_body,
        grid=(N // blk,),
        in_specs=[
            pl.BlockSpec((blk, D), lambda i: (i, 0)),
            pl.BlockSpec((blk, D), lambda i: (i, 0)),
        ],
        out_specs=pl.BlockSpec((blk, D), lambda i: (i, 0)),
        out_shape=jax.ShapeDtypeStruct((N, D), jnp.float32),
    )(p0, p1)


def _sc_aggregate(support, src, dst, adj, zeros):
    mesh = plsc.VectorSubcoreMesh(
        core_axis_name="c", subcore_axis_name="s", num_cores=NUM_CORES
    )
    cp = pltpu.CompilerParams()
    if "needs_layout_passes" in pltpu.CompilerParams.__dataclass_fields__:
        cp = dataclasses.replace(cp, needs_layout_passes=False)

    idx_buf = lambda: pltpu.VMEM((CHUNK,), jnp.int32)
    adj_buf = lambda: pltpu.VMEM((CHUNK,), jnp.float32)
    row_buf = lambda: pltpu.VMEM((CHUNK, D), jnp.float32)

    @functools.partial(
        pl.kernel,
        compiler_params=cp,
        out_type=jax.ShapeDtypeStruct((NUM_CORES, N_PAD, D), jnp.float32),
        mesh=mesh,
        scratch_types=[
            pltpu.VMEM_SHARED((N_PAD, D), jnp.float32),
            idx_buf(), idx_buf(),
            idx_buf(), idx_buf(),
            adj_buf(), adj_buf(),
            row_buf(), row_buf(),
            pltpu.SemaphoreType.DMA,
            pltpu.SemaphoreType.DMA,
        ],
    )
    def agg_kernel(
        support_hbm, src_hbm, dst_hbm, adj_hbm, zeros_hbm, out_hbm,
        acc, src0, src1, dst0, dst1, adj0, adj1, rows0, rows1, isem, gsem,
    ):
        core = lax.axis_index("c")
        sub = lax.axis_index("s")
        worker = core * NUM_SUBCORES + sub
        row0 = sub * ROWS_PER_SUBCORE
        base = worker * EDGES_PER_WORKER

        bufs = ((src0, dst0, adj0, rows0), (src1, dst1, adj1, rows1))

        def idx_start(c, b):
            sb, db, ab, _ = bufs[b]
            off = base + c * CHUNK
            pltpu.async_copy(src_hbm.at[pl.ds(off, CHUNK)], sb, isem)
            pltpu.async_copy(dst_hbm.at[pl.ds(off, CHUNK)], db, isem)
            pltpu.async_copy(adj_hbm.at[pl.ds(off, CHUNK)], ab, isem)

        def idx_wait(b):
            sb, db, ab, _ = bufs[b]
            pltpu.make_async_copy(src_hbm.at[pl.ds(0, CHUNK)], sb, isem).wait()
            pltpu.make_async_copy(dst_hbm.at[pl.ds(0, CHUNK)], db, isem).wait()
            pltpu.make_async_copy(adj_hbm.at[pl.ds(0, CHUNK)], ab, isem).wait()

        def gather_start(b):
            sb, _, _, rb = bufs[b]
            pltpu.async_copy(support_hbm.at[sb], rb, gsem)

        def gather_wait(b):
            sb, _, _, rb = bufs[b]
            pltpu.make_async_copy(support_hbm.at[sb], rb, gsem).wait()

        idx_start(0, 0)
        pltpu.sync_copy(zeros_hbm, acc.at[pl.ds(row0, ROWS_PER_SUBCORE)])
        idx_wait(0)
        gather_start(0)
        idx_start(1, 1)
        plsc.subcore_barrier()

        def compute_scatter(b):
            _, db, ab, rb = bufs[b]

            @pl.loop(0, CHUNK)
            def _(r):
                ir = lax.broadcast_in_dim(r, (LANES,), ())
                a = plsc.load_gather(ab, [ir])
                for j in range(D // LANES):
                    sl = pl.ds(j * LANES, LANES)
                    rb[r, sl] = rb[r, sl] * a

            pltpu.sync_copy(rb, acc.at[db], add=True)

        def step(c, b, ob, prefetch):
            gather_wait(b)
            idx_wait(ob)
            gather_start(ob)
            compute_scatter(b)
            if prefetch:
                idx_start(c + 2, b)

        @pl.loop(0, NUM_CHUNKS // 2 - 1)
        def _(t):
            step(t * 2, 0, 1, True)
            step(t * 2 + 1, 1, 0, True)

        step(NUM_CHUNKS - 2, 0, 1, False)
        gather_wait(1)
        compute_scatter(1)

        plsc.subcore_barrier()
        pltpu.sync_copy(
            acc.at[pl.ds(row0, ROWS_PER_SUBCORE)],
            out_hbm.at[core, pl.ds(row0, ROWS_PER_SUBCORE)],
        )

    return agg_kernel(support, src, dst, adj, zeros)


@jax.jit
def kernel(x, edge_index, adj_values, W, b):
    pad = E_PAD - E
    src = jnp.concatenate(
        [edge_index[1].astype(jnp.int32), jnp.zeros((pad,), jnp.int32)]
    )
    dst = jnp.concatenate(
        [edge_index[0].astype(jnp.int32), jnp.full((pad,), N, jnp.int32)]
    )
    adj = jnp.concatenate([adj_values, jnp.zeros((pad,), jnp.float32)])
    wt = W.T
    b2d = b.reshape(1, D)
    support = _tc_linear(x, wt, b2d)
    zeros = jnp.zeros((ROWS_PER_SUBCORE, D), jnp.float32)
    partials = _sc_aggregate(support, src, dst, adj, zeros)
    return _tc_add_tanh(partials[0, :N], partials[1, :N])


# speedup vs baseline: 1.8151x; 1.5259x over previous
import dataclasses
import functools

import jax
import jax.numpy as jnp
from jax import lax
from jax.experimental import pallas as pl
from jax.experimental.pallas import tpu as pltpu
from jax.experimental.pallas import tpu_sc as plsc

N = 10000
E = 320000
D = 128

NUM_CORES = 2
NUM_SUBCORES = 16
NUM_WORKERS = NUM_CORES * NUM_SUBCORES
EDGES_PER_WORKER = E // NUM_WORKERS
CHUNK = 200
NUM_CHUNKS = EDGES_PER_WORKER // CHUNK
N_PAD = 10240
ROWS_PER_SUBCORE = N_PAD // NUM_SUBCORES
LANES = 16


def _linear_body(x_ref, wt_ref, b_ref, out_ref):
    out_ref[...] = (
        jnp.dot(x_ref[...], wt_ref[...], preferred_element_type=jnp.float32)
        + b_ref[...]
    )


def _tc_linear(x, wt, b2d):
    return pl.pallas_call(
        _linear_body,
        out_shape=jax.ShapeDtypeStruct((N, D), jnp.float32),
    )(x, wt, b2d)


def _add_tanh_body(p0_ref, p1_ref, out_ref):
    out_ref[...] = jnp.tanh(p0_ref[...] + p1_ref[...])


def _tc_add_tanh(p0, p1):
    blk = 2000
    return pl.pallas_call(
        _add_tanh_body,
        grid=(N // blk,),
        in_specs=[
            pl.BlockSpec((blk, D), lambda i: (i, 0)),
            pl.BlockSpec((blk, D), lambda i: (i, 0)),
        ],
        out_specs=pl.BlockSpec((blk, D), lambda i: (i, 0)),
        out_shape=jax.ShapeDtypeStruct((N, D), jnp.float32),
    )(p0, p1)


def _sc_aggregate(support, src, dst, adj, zeros):
    mesh = plsc.VectorSubcoreMesh(
        core_axis_name="c", subcore_axis_name="s", num_cores=NUM_CORES
    )
    cp = pltpu.CompilerParams()
    if "needs_layout_passes" in pltpu.CompilerParams.__dataclass_fields__:
        cp = dataclasses.replace(cp, needs_layout_passes=False)

    @functools.partial(
        pl.kernel,
        compiler_params=cp,
        out_type=jax.ShapeDtypeStruct((NUM_CORES, N_PAD, D), jnp.float32),
        mesh=mesh,
        scratch_types=[
            pltpu.VMEM_SHARED((N_PAD, D), jnp.float32),
            pltpu.VMEM((CHUNK,), jnp.int32),
            pltpu.VMEM((CHUNK,), jnp.int32),
            pltpu.VMEM((CHUNK,), jnp.float32),
            pltpu.VMEM((CHUNK, D), jnp.float32),
            pltpu.SemaphoreType.DMA,
        ],
    )
    def agg_kernel(
        support_hbm, src_hbm, dst_hbm, adj_hbm, zeros_hbm, out_hbm,
        acc, src_v, dst_v, adj_v, rows_v, sem,
    ):
        core = lax.axis_index("c")
        sub = lax.axis_index("s")

        row0 = sub * ROWS_PER_SUBCORE
        pltpu.sync_copy(zeros_hbm, acc.at[pl.ds(row0, ROWS_PER_SUBCORE)])
        plsc.subcore_barrier()

        worker = core * NUM_SUBCORES + sub
        base = worker * EDGES_PER_WORKER

        @pl.loop(0, NUM_CHUNKS)
        def _(c):
            off = base + c * CHUNK
            pltpu.sync_copy(src_hbm.at[pl.ds(off, CHUNK)], src_v)
            pltpu.sync_copy(dst_hbm.at[pl.ds(off, CHUNK)], dst_v)
            pltpu.sync_copy(adj_hbm.at[pl.ds(off, CHUNK)], adj_v)
            pltpu.async_copy(support_hbm.at[src_v], rows_v, sem).wait()

            @pl.loop(0, CHUNK)
            def _(r):
                splat_idx = lax.broadcast_in_dim(r, (LANES,), ())
                a = plsc.load_gather(adj_v, [splat_idx])
                for j in range(D // LANES):
                    sl = pl.ds(j * LANES, LANES)
                    rows_v[r, sl] = rows_v[r, sl] * a

            pltpu.sync_copy(rows_v, acc.at[dst_v], add=True)

        plsc.subcore_barrier()
        pltpu.sync_copy(
            acc.at[pl.ds(row0, ROWS_PER_SUBCORE)],
            out_hbm.at[core, pl.ds(row0, ROWS_PER_SUBCORE)],
        )

    return agg_kernel(support, src, dst, adj, zeros)


@jax.jit
def kernel(x, edge_index, adj_values, W, b):
    src = edge_index[1].astype(jnp.int32)
    dst = edge_index[0].astype(jnp.int32)
    wt = W.T
    b2d = b.reshape(1, D)
    support = _tc_linear(x, wt, b2d)
    zeros = jnp.zeros((ROWS_PER_SUBCORE, D), jnp.float32)
    partials = _sc_aggregate(support, src, dst, adj_values, zeros)
    return _tc_add_tanh(partials[0, :N], partials[1, :N])


# speedup vs baseline: 2.0556x; 1.1325x over previous
import dataclasses
import functools

import jax
import jax.numpy as jnp
from jax import lax
from jax.experimental import pallas as pl
from jax.experimental.pallas import tpu as pltpu
from jax.experimental.pallas import tpu_sc as plsc

N = 10000
E = 320000
D = 128

NUM_CORES = 2
NUM_SUBCORES = 16
NUM_WORKERS = NUM_CORES * NUM_SUBCORES
EDGES_PER_WORKER = E // NUM_WORKERS
CHUNK = 200
NUM_CHUNKS = EDGES_PER_WORKER // CHUNK
N_PAD = 10240
ROWS_PER_SUBCORE = N_PAD // NUM_SUBCORES
LANES = 16


def _linear_body(x_ref, wt_ref, b_ref, out_ref):
    out_ref[...] = (
        jnp.dot(x_ref[...], wt_ref[...], preferred_element_type=jnp.float32)
        + b_ref[...]
    )


def _tc_linear(x, wt, b2d):
    return pl.pallas_call(
        _linear_body,
        out_shape=jax.ShapeDtypeStruct((N, D), jnp.float32),
    )(x, wt, b2d)


def _add_tanh_body(p0_ref, p1_ref, out_ref):
    out_ref[...] = jnp.tanh(p0_ref[...] + p1_ref[...])


def _tc_add_tanh(p0, p1):
    blk = 2000
    return pl.pallas_call(
        _add_tanh_body,
        grid=(N // blk,),
        in_specs=[
            pl.BlockSpec((blk, D), lambda i: (i, 0)),
            pl.BlockSpec((blk, D), lambda i: (i, 0)),
        ],
        out_specs=pl.BlockSpec((blk, D), lambda i: (i, 0)),
        out_shape=jax.ShapeDtypeStruct((N, D), jnp.float32),
    )(p0, p1)


def _sc_aggregate(support, src, dst, adj, zeros):
    mesh = plsc.VectorSubcoreMesh(
        core_axis_name="c", subcore_axis_name="s", num_cores=NUM_CORES
    )
    cp = pltpu.CompilerParams()
    if "needs_layout_passes" in pltpu.CompilerParams.__dataclass_fields__:
        cp = dataclasses.replace(cp, needs_layout_passes=False)

    @functools.partial(
        pl.kernel,
        compiler_params=cp,
        out_type=jax.ShapeDtypeStruct((NUM_CORES, N_PAD, D), jnp.float32),
        mesh=mesh,
        scratch_types=[
            pltpu.VMEM_SHARED((N_PAD, D), jnp.float32),
            pltpu.VMEM((CHUNK,), jnp.int32),
            pltpu.VMEM((CHUNK,), jnp.int32),
            pltpu.VMEM((CHUNK,), jnp.float32),
            pltpu.VMEM((CHUNK, D), jnp.float32),
            pltpu.SemaphoreType.DMA,
            pltpu.SemaphoreType.DMA,
        ],
    )
    def agg_kernel(
        support_hbm, src_hbm, dst_hbm, adj_hbm, zeros_hbm, out_hbm,
        acc, src_v, dst_v, adj_v, rows_v, sem, isem,
    ):
        core = lax.axis_index("c")
        sub = lax.axis_index("s")

        row0 = sub * ROWS_PER_SUBCORE
        pltpu.sync_copy(zeros_hbm, acc.at[pl.ds(row0, ROWS_PER_SUBCORE)])
        plsc.subcore_barrier()

        worker = core * NUM_SUBCORES + sub
        base = worker * EDGES_PER_WORKER

        @pl.loop(0, NUM_CHUNKS)
        def _(c):
            off = base + c * CHUNK
            d0 = pltpu.async_copy(src_hbm.at[pl.ds(off, CHUNK)], src_v, isem)
            d1 = pltpu.async_copy(dst_hbm.at[pl.ds(off, CHUNK)], dst_v, isem)
            d2 = pltpu.async_copy(adj_hbm.at[pl.ds(off, CHUNK)], adj_v, isem)
            d0.wait()
            d1.wait()
            d2.wait()
            pltpu.async_copy(support_hbm.at[src_v], rows_v, sem).wait()

            @pl.loop(0, CHUNK)
            def _(r):
                splat_idx = lax.broadcast_in_dim(r, (LANES,), ())
                a = plsc.load_gather(adj_v, [splat_idx])
                for j in range(D // LANES):
                    sl = pl.ds(j * LANES, LANES)
                    rows_v[r, sl] = rows_v[r, sl] * a

            pltpu.sync_copy(rows_v, acc.at[dst_v], add=True)

        plsc.subcore_barrier()
        pltpu.sync_copy(
            acc.at[pl.ds(row0, ROWS_PER_SUBCORE)],
            out_hbm.at[core, pl.ds(row0, ROWS_PER_SUBCORE)],
        )

    return agg_kernel(support, src, dst, adj, zeros)


@jax.jit
def kernel(x, edge_index, adj_values, W, b):
    src = edge_index[1].astype(jnp.int32)
    dst = edge_index[0].astype(jnp.int32)
    wt = W.T
    b2d = b.reshape(1, D)
    support = _tc_linear(x, wt, b2d)
    zeros = jnp.zeros((ROWS_PER_SUBCORE, D), jnp.float32)
    partials = _sc_aggregate(support, src, dst, adj_values, zeros)
    return _tc_add_tanh(partials[0, :N], partials[1, :N])
